# baseline (speedup 1.0000x reference)
import jax
import jax.numpy as jnp
from jax import lax
from jax.experimental import pallas as pl
from jax.experimental.pallas import tpu as pltpu

_L = 2
_H = 2
_F_IN = 128
_F_OUT = 64
_N = 4096
_R = 256
_NB = _N // _R


def _layer_body(h_ref, adj_ref, w_ref, as_ref, an_ref, b_ref,
                out_ref, adjb_ref, feat_scr, ext_scr, en_scr, fnmax_scr,
                *, first_layer):
    i = pl.program_id(1)

    a = adj_ref[0]
    if first_layer:
        af = a.astype(jnp.bfloat16)
        adjb_ref[0] = af
    else:
        af = a

    @pl.when(i == 0)
    def _per_graph_prologue():
        hfull = h_ref[0]
        onescol = jnp.where(
            lax.broadcasted_iota(jnp.int32, (_N, _F_OUT), 1) == 0, 1.0, 0.0)
        for hd in range(_H):
            feat = jnp.dot(hfull, w_ref[hd],
                           preferred_element_type=jnp.float32)
            feat_scr[hd] = feat
            ext_scr[hd] = jnp.concatenate(
                [feat, onescol], axis=1).astype(jnp.bfloat16)
            anl = an_ref[hd]
            fn = lax.dot_general(anl, feat, (((1,), (1,)), ((), ())),
                                 preferred_element_type=jnp.float32)
            fnmax = jnp.max(fn)
            fnmax_scr[hd] = fnmax
            en_scr[hd, 0:1, :] = jnp.exp(fn - fnmax)
            en_scr[hd, 1:2, :] = jnp.exp(0.2 * (fn - fnmax))

    outs = []
    for hd in range(_H):
        feat_blk = feat_scr[hd, pl.ds(i * _R, _R), :]
        fs = jnp.dot(feat_blk, as_ref[hd],
                     preferred_element_type=jnp.float32)
        fnmax = fnmax_scr[hd]
        xmax = fs + fnmax
        s = jnp.where(xmax >= 0.0, xmax, 0.2 * xmax)
        ea = jnp.exp(xmax - s)
        eb = jnp.exp(0.2 * xmax - s)
        en = en_scr[hd, 0:1, :]
        en2 = en_scr[hd, 1:2, :]
        p = af * jnp.maximum(ea * en, eb * en2).astype(jnp.bfloat16)
        o2 = jnp.dot(p, ext_scr[hd],
                     preferred_element_type=jnp.float32)
        rowsum = o2[:, _F_OUT:_F_OUT + 1]
        outs.append(o2[:, 0:_F_OUT] / rowsum + b_ref[hd])

    out = jnp.concatenate(outs, axis=-1)
    out_ref[0] = jnp.where(out > 0.0, out, jnp.exp(out) - 1.0)


def _run_layer(h, adj, Wl, asl, anl, bl, *, first_layer):
    B = h.shape[0]
    grid = (B, _NB)
    in_specs = [
        pl.BlockSpec((1, _N, _F_IN), lambda g, i: (g, 0, 0)),
        pl.BlockSpec((1, _R, _N), lambda g, i: (g, i, 0)),
        pl.BlockSpec((_H, _F_IN, _F_OUT), lambda g, i: (0, 0, 0)),
        pl.BlockSpec((_H, _F_OUT, 1), lambda g, i: (0, 0, 0)),
        pl.BlockSpec((_H, 1, _F_OUT), lambda g, i: (0, 0, 0)),
        pl.BlockSpec((_H, 1, _F_OUT), lambda g, i: (0, 0, 0)),
    ]
    out_spec_h = pl.BlockSpec((1, _R, _H * _F_OUT), lambda g, i: (g, i, 0))
    out_shape_h = jax.ShapeDtypeStruct((B, _N, _H * _F_OUT), jnp.float32)
    if first_layer:
        out_specs = [out_spec_h,
                     pl.BlockSpec((1, _R, _N), lambda g, i: (g, i, 0))]
        out_shapes = [out_shape_h,
                      jax.ShapeDtypeStruct((B, _N, _N), jnp.bfloat16)]

        def body(h_ref, adj_ref, w_ref, as_ref, an_ref, b_ref, out_ref,
                 adjb_ref, feat_scr, ext_scr, en_scr, fnmax_scr):
            _layer_body(h_ref, adj_ref, w_ref, as_ref, an_ref, b_ref,
                        out_ref, adjb_ref, feat_scr, ext_scr, en_scr,
                        fnmax_scr, first_layer=True)
    else:
        out_specs = [out_spec_h]
        out_shapes = [out_shape_h]

        def body(h_ref, adj_ref, w_ref, as_ref, an_ref, b_ref, out_ref,
                 feat_scr, ext_scr, en_scr, fnmax_scr):
            _layer_body(h_ref, adj_ref, w_ref, as_ref, an_ref, b_ref,
                        out_ref, None, feat_scr, ext_scr, en_scr,
                        fnmax_scr, first_layer=False)

    return pl.pallas_call(
        body,
        grid=grid,
        in_specs=in_specs,
        out_specs=out_specs,
        out_shape=out_shapes,
        scratch_shapes=[
            pltpu.VMEM((_H, _N, _F_OUT), jnp.float32),
            pltpu.VMEM((_H, _N, 2 * _F_OUT), jnp.bfloat16),
            pltpu.VMEM((_H, 2, _N), jnp.float32),
            pltpu.SMEM((_H,), jnp.float32),
        ],
    )(h, adj, Wl, asl[:, :, None], anl[:, None, :], bl[:, None, :])


def kernel(x, adj, W, a_self, a_neigh, b):
    h, adjb = _run_layer(x, adj, W[0], a_self[0], a_neigh[0], b[0],
                         first_layer=True)
    for l in range(1, _L):
        (h,) = _run_layer(h, adjb, W[l], a_self[l], a_neigh[l], b[l],
                          first_layer=False)
    return h


# speedup vs baseline: 1.1886x; 1.1886x over previous
import jax
import jax.numpy as jnp
from jax import lax
from jax.experimental import pallas as pl
from jax.experimental.pallas import tpu as pltpu

_L = 2
_H = 2
_F_IN = 128
_F_OUT = 64
_N = 4096
_R = 256
_NB = _N // _R


def _layer_body(h_ref, adj_ref, w_ref, as_ref, an_ref, b_ref,
                out_ref, adjb_ref, fs_scr, ext_scr, en_scr, fnmax_scr,
                *, first_layer):
    i = pl.program_id(1)

    a = adj_ref[0]
    if first_layer:
        af = a.astype(jnp.bfloat16)
        adjb_ref[0] = af
    else:
        af = a

    @pl.when(i == 0)
    def _per_graph_prologue():
        hfull = h_ref[0]
        onescol = jnp.where(
            lax.broadcasted_iota(jnp.int32, (_N, _F_OUT), 1) == 0, 1.0, 0.0)
        for hd in range(_H):
            feat = jnp.dot(hfull, w_ref[hd],
                           preferred_element_type=jnp.float32)
            ext_scr[hd] = jnp.concatenate(
                [feat, onescol], axis=1).astype(jnp.bfloat16)
            anl = an_ref[hd]
            fn = lax.dot_general(anl, feat, (((1,), (1,)), ((), ())),
                                 preferred_element_type=jnp.float32)
            fnmax = jnp.max(fn)
            fnmax_scr[hd] = fnmax
            en_scr[hd, 0:1, :] = jnp.exp(fn - fnmax).astype(jnp.bfloat16)
            en_scr[hd, 1:2, :] = jnp.exp(
                0.2 * (fn - fnmax)).astype(jnp.bfloat16)
            fs_scr[hd] = jnp.dot(feat, as_ref[hd],
                                 preferred_element_type=jnp.float32)

    outs = []
    for hd in range(_H):
        fs = fs_scr[hd, pl.ds(i * _R, _R), :]
        fnmax = fnmax_scr[hd]
        xmax = fs + fnmax
        s = jnp.where(xmax >= 0.0, xmax, 0.2 * xmax)
        ea = jnp.exp(xmax - s).astype(jnp.bfloat16)
        eb = jnp.exp(0.2 * xmax - s).astype(jnp.bfloat16)
        en = en_scr[hd, 0:1, :]
        en2 = en_scr[hd, 1:2, :]
        p = af * jnp.maximum(ea * en, eb * en2)
        o2 = jnp.dot(p, ext_scr[hd],
                     preferred_element_type=jnp.float32)
        rowsum = o2[:, _F_OUT:_F_OUT + 1]
        outs.append(o2[:, 0:_F_OUT] / rowsum + b_ref[hd])

    out = jnp.concatenate(outs, axis=-1)
    out_ref[0] = jnp.where(out > 0.0, out, jnp.exp(out) - 1.0)


def _run_layer(h, adj, Wl, asl, anl, bl, *, first_layer):
    B = h.shape[0]
    grid = (B, _NB)
    in_specs = [
        pl.BlockSpec((1, _N, _F_IN), lambda g, i: (g, 0, 0)),
        pl.BlockSpec((1, _R, _N), lambda g, i: (g, i, 0)),
        pl.BlockSpec((_H, _F_IN, _F_OUT), lambda g, i: (0, 0, 0)),
        pl.BlockSpec((_H, _F_OUT, 1), lambda g, i: (0, 0, 0)),
        pl.BlockSpec((_H, 1, _F_OUT), lambda g, i: (0, 0, 0)),
        pl.BlockSpec((_H, 1, _F_OUT), lambda g, i: (0, 0, 0)),
    ]
    out_spec_h = pl.BlockSpec((1, _R, _H * _F_OUT), lambda g, i: (g, i, 0))
    out_shape_h = jax.ShapeDtypeStruct((B, _N, _H * _F_OUT), jnp.float32)
    if first_layer:
        out_specs = [out_spec_h,
                     pl.BlockSpec((1, _R, _N), lambda g, i: (g, i, 0))]
        out_shapes = [out_shape_h,
                      jax.ShapeDtypeStruct((B, _N, _N), jnp.bfloat16)]

        def body(h_ref, adj_ref, w_ref, as_ref, an_ref, b_ref, out_ref,
                 adjb_ref, fs_scr, ext_scr, en_scr, fnmax_scr):
            _layer_body(h_ref, adj_ref, w_ref, as_ref, an_ref, b_ref,
                        out_ref, adjb_ref, fs_scr, ext_scr, en_scr,
                        fnmax_scr, first_layer=True)
    else:
        out_specs = [out_spec_h]
        out_shapes = [out_shape_h]

        def body(h_ref, adj_ref, w_ref, as_ref, an_ref, b_ref, out_ref,
                 fs_scr, ext_scr, en_scr, fnmax_scr):
            _layer_body(h_ref, adj_ref, w_ref, as_ref, an_ref, b_ref,
                        out_ref, None, fs_scr, ext_scr, en_scr,
                        fnmax_scr, first_layer=False)

    return pl.pallas_call(
        body,
        grid=grid,
        in_specs=in_specs,
        out_specs=out_specs,
        out_shape=out_shapes,
        scratch_shapes=[
            pltpu.VMEM((_H, _N, 1), jnp.float32),
            pltpu.VMEM((_H, _N, 2 * _F_OUT), jnp.bfloat16),
            pltpu.VMEM((_H, 2, _N), jnp.bfloat16),
            pltpu.SMEM((_H,), jnp.float32),
        ],
    )(h, adj, Wl, asl[:, :, None], anl[:, None, :], bl[:, None, :])


def kernel(x, adj, W, a_self, a_neigh, b):
    h, adjb = _run_layer(x, adj, W[0], a_self[0], a_neigh[0], b[0],
                         first_layer=True)
    for l in range(1, _L):
        (h,) = _run_layer(h, adjb, W[l], a_self[l], a_neigh[l], b[l],
                          first_layer=False)
    return h


# speedup vs baseline: 1.3859x; 1.1659x over previous
import jax
import jax.numpy as jnp
from jax import lax
from jax.experimental import pallas as pl
from jax.experimental.pallas import tpu as pltpu

_L = 2
_H = 2
_F_IN = 128
_F_OUT = 64
_N = 4096
_R = 256
_NB = _N // _R


def _body(h_ref, adj_ref, w_ref, as_ref, an_ref, b_ref, out_ref,
          adjb_scr, h1_scr, fs_scr, ext_scr, en_scr, fnmax_scr):
    l = pl.program_id(1)
    i = pl.program_id(2)
    first_layer = l == 0

    @pl.when(first_layer & (i == 0))
    def _stage_input():
        h1_scr[...] = h_ref[0]

    @pl.when(i == 0)
    def _per_graph_layer_prologue():
        hfull = h1_scr[...]
        onescol = jnp.where(
            lax.broadcasted_iota(jnp.int32, (_N, _F_OUT), 1) == 0, 1.0, 0.0)
        for hd in range(_H):
            feat = jnp.dot(hfull, w_ref[l, hd],
                           preferred_element_type=jnp.float32)
            ext_scr[hd] = jnp.concatenate(
                [feat, onescol], axis=1).astype(jnp.bfloat16)
            anl = an_ref[l, hd]
            fn = lax.dot_general(anl, feat, (((1,), (1,)), ((), ())),
                                 preferred_element_type=jnp.float32)
            fnmax = jnp.max(fn)
            fnmax_scr[hd] = fnmax
            en_scr[hd, 0:1, :] = jnp.exp(fn - fnmax).astype(jnp.bfloat16)
            en_scr[hd, 1:2, :] = jnp.exp(
                0.2 * (fn - fnmax)).astype(jnp.bfloat16)
            fs_scr[hd] = jnp.dot(feat, as_ref[l, hd],
                                 preferred_element_type=jnp.float32)

    def _attend(af):
        outs = []
        for hd in range(_H):
            fs = fs_scr[hd, pl.ds(i * _R, _R), :]
            fnmax = fnmax_scr[hd]
            xmax = fs + fnmax
            s = jnp.where(xmax >= 0.0, xmax, 0.2 * xmax)
            ea = jnp.exp(xmax - s).astype(jnp.bfloat16)
            eb = jnp.exp(0.2 * xmax - s).astype(jnp.bfloat16)
            en = en_scr[hd, 0:1, :]
            en2 = en_scr[hd, 1:2, :]
            p = af * jnp.maximum(ea * en, eb * en2)
            o2 = jnp.dot(p, ext_scr[hd],
                         preferred_element_type=jnp.float32)
            rowsum = o2[:, _F_OUT:_F_OUT + 1]
            outs.append(o2[:, 0:_F_OUT] / rowsum + b_ref[l, hd])
        out = jnp.concatenate(outs, axis=-1)
        return jnp.where(out > 0.0, out, jnp.exp(out) - 1.0)

    @pl.when(first_layer)
    def _layer0():
        af = adj_ref[0].astype(jnp.bfloat16)
        adjb_scr[i] = af
        h1_scr[pl.ds(i * _R, _R), :] = _attend(af)

    @pl.when(jnp.logical_not(first_layer))
    def _layer1():
        out_ref[0] = _attend(adjb_scr[i])


def kernel(x, adj, W, a_self, a_neigh, b):
    B = x.shape[0]
    grid = (B, _L, _NB)
    last = _NB - 1
    in_specs = [
        pl.BlockSpec((1, _N, _F_IN), lambda g, l, i: (g, 0, 0)),
        pl.BlockSpec((1, _R, _N),
                     lambda g, l, i: (g, jnp.where(l == 0, i, last), 0)),
        pl.BlockSpec((_L, _H, _F_IN, _F_OUT), lambda g, l, i: (0, 0, 0, 0)),
        pl.BlockSpec((_L, _H, _F_OUT, 1), lambda g, l, i: (0, 0, 0, 0)),
        pl.BlockSpec((_L, _H, 1, _F_OUT), lambda g, l, i: (0, 0, 0, 0)),
        pl.BlockSpec((_L, _H, 1, _F_OUT), lambda g, l, i: (0, 0, 0, 0)),
    ]
    out_specs = pl.BlockSpec(
        (1, _R, _H * _F_OUT), lambda g, l, i: (g, jnp.where(l == 0, 0, i), 0))
    out_shape = jax.ShapeDtypeStruct((B, _N, _H * _F_OUT), jnp.float32)

    return pl.pallas_call(
        _body,
        grid=grid,
        in_specs=in_specs,
        out_specs=out_specs,
        out_shape=out_shape,
        scratch_shapes=[
            pltpu.VMEM((_NB, _R, _N), jnp.bfloat16),
            pltpu.VMEM((_N, _F_IN), jnp.float32),
            pltpu.VMEM((_H, _N, 1), jnp.float32),
            pltpu.VMEM((_H, _N, 2 * _F_OUT), jnp.bfloat16),
            pltpu.VMEM((_H, 2, _N), jnp.bfloat16),
            pltpu.SMEM((_H,), jnp.float32),
        ],
    )(x, adj, W, a_self[:, :, :, None], a_neigh[:, :, None, :],
      b[:, :, None, :])


# speedup vs baseline: 1.4236x; 1.0273x over previous
import jax
import jax.numpy as jnp
from jax import lax
from jax.experimental import pallas as pl
from jax.experimental.pallas import tpu as pltpu

_L = 2
_H = 2
_F_IN = 128
_F_OUT = 64
_N = 4096
_R = 256
_NB = _N // _R


def _body(h_ref, adj_ref, w_ref, as_ref, an_ref, b_ref, out_ref,
          adjb_scr, h1_scr, fs_scr, ext_scr, en_scr, fnmax_scr):
    l = pl.program_id(1)
    i = pl.program_id(2)
    first_layer = l == 0

    @pl.when(first_layer & (i == 0))
    def _stage_input():
        h1_scr[...] = h_ref[0]

    @pl.when(first_layer & (i == 0) & (pl.program_id(0) == 0))
    def _init_ones_cols():
        onescol = jnp.where(
            lax.broadcasted_iota(jnp.int32, (_N, _F_OUT), 1) == 0,
            1.0, 0.0).astype(jnp.bfloat16)
        for hd in range(_H):
            ext_scr[hd, :, _F_OUT:2 * _F_OUT] = onescol

    @pl.when(i == 0)
    def _per_graph_layer_prologue():
        hfull = h1_scr[...]
        for hd in range(_H):
            feat = jnp.dot(hfull, w_ref[l, hd],
                           preferred_element_type=jnp.float32)
            ext_scr[hd, :, 0:_F_OUT] = feat.astype(jnp.bfloat16)
            anl = an_ref[l, hd]
            fn = lax.dot_general(anl, feat, (((1,), (1,)), ((), ())),
                                 preferred_element_type=jnp.float32)
            fnmax = jnp.max(fn)
            fnmax_scr[hd] = fnmax
            en_scr[hd, 0:1, :] = jnp.exp(fn - fnmax).astype(jnp.bfloat16)
            en_scr[hd, 1:2, :] = jnp.exp(
                0.2 * (fn - fnmax)).astype(jnp.bfloat16)
            fs_scr[hd] = jnp.dot(feat, as_ref[l, hd],
                                 preferred_element_type=jnp.float32)

    def _attend(af):
        outs = []
        for hd in range(_H):
            fs = fs_scr[hd, pl.ds(i * _R, _R), :]
            fnmax = fnmax_scr[hd]
            xmax = fs + fnmax
            r = jnp.exp(-0.8 * xmax).astype(jnp.bfloat16)
            en = en_scr[hd, 0:1, :]
            en2 = en_scr[hd, 1:2, :]
            p = af * jnp.maximum(en, r * en2)
            o2 = jnp.dot(p, ext_scr[hd],
                         preferred_element_type=jnp.float32)
            rowsum = o2[:, _F_OUT:_F_OUT + 1]
            outs.append(o2[:, 0:_F_OUT] / rowsum + b_ref[l, hd])
        out = jnp.concatenate(outs, axis=-1)
        return jnp.where(out > 0.0, out, jnp.exp(out) - 1.0)

    @pl.when(first_layer)
    def _layer0():
        af = adj_ref[0].astype(jnp.bfloat16)
        adjb_scr[i] = af
        h1_scr[pl.ds(i * _R, _R), :] = _attend(af)

    @pl.when(jnp.logical_not(first_layer))
    def _layer1():
        out_ref[0] = _attend(adjb_scr[i])


def kernel(x, adj, W, a_self, a_neigh, b):
    B = x.shape[0]
    grid = (B, _L, _NB)
    last = _NB - 1
    in_specs = [
        pl.BlockSpec((1, _N, _F_IN), lambda g, l, i: (g, 0, 0)),
        pl.BlockSpec((1, _R, _N),
                     lambda g, l, i: (g, jnp.where(l == 0, i, last), 0)),
        pl.BlockSpec((_L, _H, _F_IN, _F_OUT), lambda g, l, i: (0, 0, 0, 0)),
        pl.BlockSpec((_L, _H, _F_OUT, 1), lambda g, l, i: (0, 0, 0, 0)),
        pl.BlockSpec((_L, _H, 1, _F_OUT), lambda g, l, i: (0, 0, 0, 0)),
        pl.BlockSpec((_L, _H, 1, _F_OUT), lambda g, l, i: (0, 0, 0, 0)),
    ]
    out_specs = pl.BlockSpec(
        (1, _R, _H * _F_OUT), lambda g, l, i: (g, jnp.where(l == 0, 0, i), 0))
    out_shape = jax.ShapeDtypeStruct((B, _N, _H * _F_OUT), jnp.float32)

    return pl.pallas_call(
        _body,
        grid=grid,
        in_specs=in_specs,
        out_specs=out_specs,
        out_shape=out_shape,
        scratch_shapes=[
            pltpu.VMEM((_NB, _R, _N), jnp.bfloat16),
            pltpu.VMEM((_N, _F_IN), jnp.float32),
            pltpu.VMEM((_H, _N, 1), jnp.float32),
            pltpu.VMEM((_H, _N, 2 * _F_OUT), jnp.bfloat16),
            pltpu.VMEM((_H, 2, _N), jnp.bfloat16),
            pltpu.SMEM((_H,), jnp.float32),
        ],
    )(x, adj, W, a_self[:, :, :, None], a_neigh[:, :, None, :],
      b[:, :, None, :])
